# baseline (speedup 1.0000x reference)
import functools
import math

import jax
import jax.numpy as jnp
import numpy as np
from jax import lax
from jax.experimental import pallas as pl
from jax.experimental.pallas import tpu as pltpu
from jax.experimental.pallas import tpu_sc as plsc

N_NODES = 10000
E = 160000
D_X = 128
MUL = 32
MOD_DIM = 16
RANK = 4
ALPHA = 8.0
HID = 128

C = 128
BE = 6400
GRID = E // BE

NC = 2
NT = 16
B = 128
NCHK = E // B
NPAIR = NCHK // 2
PE = 2 * B
ACC_N = 10240

_SQ3 = math.sqrt(3.0)
_D0 = (((0,), (0,)), ((), ()))

_EXP32 = np.zeros((MUL, 3 * MUL), np.float32)
_TILE3 = np.zeros((3, 3 * MUL), np.float32)
_RED3 = np.zeros((3 * MUL, MUL), np.float32)
for _c in range(MUL):
    for _i in range(3):
        _EXP32[_c, 3 * _c + _i] = 1.0
        _TILE3[_i, 3 * _c + _i] = 1.0
        _RED3[3 * _c + _i, _c] = 1.0
_K42 = np.array([[1, 0], [0, 1], [0, 1], [0, 1]], np.float32)
_EYE3 = np.eye(3, dtype=np.float32)


def _pre_kernel(x_ref, m_ref, vt_ref, md_ref, w1f_ref, a1_ref,
                b1f_ref, kp_ref, vals_ref):
    vt = vt_ref[...]
    stack = jnp.concatenate([m_ref[...], vt * vt, vt, md_ref[...]],
                            axis=0)
    cols = lax.dot_general(stack, kp_ref[...], _D0,
                           precision=lax.Precision.HIGHEST,
                           preferred_element_type=jnp.float32)
    vtil = cols[:, 0:96]
    m = cols[:, 96:97]
    scal = (_SQ3 * m) / (jnp.sqrt(cols[:, 97:98]) + 1e-12)
    s1 = cols[:, 98:102]
    xm = x_ref[...] * m
    t = jnp.dot(xm, a1_ref[...], preferred_element_type=jnp.float32) * s1
    w = (jnp.dot(xm, w1f_ref[...], preferred_element_type=jnp.float32)
         + jnp.dot(t, b1f_ref[...], preferred_element_type=jnp.float32))
    vals_ref[:, 0:MUL] = w[:, 0:MUL] * m
    vals_ref[:, MUL:C] = w[:, MUL:C] * vtil * scal


def _post_kernel(x_ref, m_ref, u_ref, nnb_ref, sm_ref, v_ref, md_ref,
                 w2a_ref, w2b_ref, a2a_ref, a2b_ref,
                 b2_ref,
                 w3_ref, a3_ref, b3_ref,
                 w4_ref, a4_ref, b4_ref,
                 p1_ref, redp_ref, wtop_ref, wbot_ref, ks_ref,
                 xout_ref, vout_ref):
    stack = jnp.concatenate([m_ref[...], u_ref[...], nnb_ref[...],
                             md_ref[...]], axis=0)
    cols = lax.dot_general(stack, ks_ref[...], _D0,
                           precision=lax.Precision.HIGHEST,
                           preferred_element_type=jnp.float32)
    m = cols[:, 0:1]
    u = cols[:, 1:2]
    inv = 1.0 / (cols[:, 2:3] + 1e-5)
    s2 = cols[:, 3:7]
    s3 = cols[:, 7:11]
    s4 = cols[:, 11:15]
    swvw = sm_ref[...] * inv
    v = v_ref[...]
    v_v = v[:, MUL:]
    pv = swvw * v
    out0ab = jnp.dot(pv, redp_ref[...], preferred_element_type=jnp.float32)
    sw_e = jnp.dot(swvw, p1_ref[...], preferred_element_type=jnp.float32)
    vs_e = jnp.dot(v, p1_ref[...], preferred_element_type=jnp.float32)
    vout = (jnp.dot(sw_e * v_v, wtop_ref[...], preferred_element_type=jnp.float32)
            + jnp.dot(swvw[:, MUL:] * vs_e, wbot_ref[...],
                      preferred_element_type=jnp.float32))
    xm = x_ref[...] * m

    t2 = (jnp.dot(xm, a2a_ref[...], preferred_element_type=jnp.float32)
          + jnp.dot(out0ab, a2b_ref[...], preferred_element_type=jnp.float32)
          ) * s2
    h = (jnp.dot(xm, w2a_ref[...], preferred_element_type=jnp.float32)
         + jnp.dot(out0ab, w2b_ref[...], preferred_element_type=jnp.float32)
         + jnp.dot(t2, b2_ref[...], preferred_element_type=jnp.float32))
    h = h * jax.nn.sigmoid(h)

    def lora(hh, w_r, a_r, b_r, s):
        t = jnp.dot(hh, a_r[...], preferred_element_type=jnp.float32) * s
        return (jnp.dot(hh, w_r[...], preferred_element_type=jnp.float32)
                + jnp.dot(t, b_r[...], preferred_element_type=jnp.float32))

    h = lora(h, w3_ref, a3_ref, b3_ref, s3)
    h = h * jax.nn.sigmoid(h)
    h = lora(h, w4_ref, a4_ref, b4_ref, s4)
    xout_ref[...] = u * h
    vout_ref[...] = vout


def _seg_body(vals_hbm, m_hbm, snd_hbm, zb2_hbm, zb1_hbm,
              out_hbm, outm_hbm,
              idx2, buf2, mbuf2, acc, accm,
              semi, semv, semm, semg, semw, semwm):
    tile = lax.axis_index("s")
    core = lax.axis_index("c")
    rpt = ACC_N // NT

    for k in range(rpt // B):
        pltpu.sync_copy(zb2_hbm, acc.at[pl.ds(tile * rpt + k * B, B)])
        pltpu.sync_copy(zb1_hbm, accm.at[pl.ds(tile * rpt + k * B, B)])
    plsc.subcore_barrier()

    base_p = NCHK // NT
    remp = NCHK - base_p * NT
    p0 = tile * base_p + jnp.minimum(tile, remp)
    nch = base_p + jnp.where(tile < remp, 1, 0)

    def start_loads(p, b):
        pltpu.async_copy(snd_hbm.at[p], idx2.at[b], semi.at[b])
        pltpu.async_copy(vals_hbm.at[pl.ds(p * B, B)], buf2.at[b], semv.at[b])
        pltpu.async_copy(m_hbm.at[pl.ds(p * B, B)], mbuf2.at[b], semm.at[b])

    start_loads(p0, 0)

    def sbody(g, carry):
        b = g % 2
        pltpu.make_async_copy(snd_hbm.at[0], idx2.at[b], semi.at[b]).wait()
        pltpu.make_async_copy(vals_hbm.at[pl.ds(0, B)], buf2.at[b],
                              semv.at[b]).wait()
        pltpu.make_async_copy(m_hbm.at[pl.ds(0, B)], mbuf2.at[b],
                              semm.at[b]).wait()
        pl.when(g + 1 < nch)(lambda: start_loads(p0 + g + 1, (g + 1) % 2))
        pltpu.sync_copy(buf2.at[b], acc.at[idx2.at[b]], add=True)
        pltpu.sync_copy(mbuf2.at[b], accm.at[idx2.at[b]], add=True)
        return carry

    lax.fori_loop(0, nch, sbody, 0)
    plsc.subcore_barrier()

    w = core * NT + tile
    nw = NC * NT
    base_q = NCHK // nw
    rem = NCHK - base_q * nw
    q0 = w * base_q + jnp.minimum(w, rem)
    nq = base_q + jnp.where(w < rem, 1, 0)

    pltpu.async_copy(snd_hbm.at[q0], idx2.at[0], semi.at[0])

    def gbody(g, carry):
        b = g % 2
        p = q0 + g
        pltpu.make_async_copy(snd_hbm.at[0], idx2.at[b], semi.at[b]).wait()

        def fetch_next():
            pltpu.async_copy(snd_hbm.at[p + 1], idx2.at[(g + 1) % 2],
                             semi.at[(g + 1) % 2])

        pl.when(g + 1 < nq)(fetch_next)

        def wait_writes():
            pltpu.make_async_copy(buf2.at[b], out_hbm.at[pl.ds(0, B)],
                                  semw.at[b]).wait()
            pltpu.make_async_copy(mbuf2.at[b], outm_hbm.at[pl.ds(0, B)],
                                  semwm.at[b]).wait()

        pl.when(g >= 2)(wait_writes)
        pltpu.async_copy(acc.at[idx2.at[b]], buf2.at[b], semg).wait()
        pltpu.async_copy(accm.at[idx2.at[b]], mbuf2.at[b], semg).wait()
        pltpu.async_copy(buf2.at[b], out_hbm.at[pl.ds(p * B, B)], semw.at[b])
        pltpu.async_copy(mbuf2.at[b], outm_hbm.at[pl.ds(p * B, B)],
                         semwm.at[b])
        return carry

    lax.fori_loop(0, nq, gbody, 0)
    for b in range(2):
        pltpu.make_async_copy(buf2.at[b], out_hbm.at[pl.ds(0, B)],
                              semw.at[b]).wait()
        pltpu.make_async_copy(mbuf2.at[b], outm_hbm.at[pl.ds(0, B)],
                              semwm.at[b]).wait()


@functools.cache
def _seg_call():
    return functools.partial(
        pl.kernel,
        out_type=[
            jax.ShapeDtypeStruct((E, C), jnp.float32),
            jax.ShapeDtypeStruct((E,), jnp.float32),
        ],
        mesh=plsc.VectorSubcoreMesh(core_axis_name="c", subcore_axis_name="s",
                                    num_cores=NC, num_subcores=NT),
        scratch_types=[
            pltpu.VMEM((2, B), jnp.int32),
            pltpu.VMEM((2, B, C), jnp.float32),
            pltpu.VMEM((2, B), jnp.float32),
            pltpu.VMEM_SHARED((ACC_N, C), jnp.float32),
            pltpu.VMEM_SHARED((ACC_N,), jnp.float32),
            pltpu.SemaphoreType.DMA((2,)),
            pltpu.SemaphoreType.DMA((2,)),
            pltpu.SemaphoreType.DMA((2,)),
            pltpu.SemaphoreType.DMA,
            pltpu.SemaphoreType.DMA((2,)),
            pltpu.SemaphoreType.DMA((2,)),
        ],
    )(_seg_body)


def _edge_spec(d):
    return pl.BlockSpec((BE, d), lambda i: (i, 0))


def _thin_spec(k):
    return pl.BlockSpec((k, BE), lambda i: (0, i))


def _full_spec(shape):
    nd = len(shape)
    return pl.BlockSpec(shape, lambda i: (0,) * nd)


def kernel(vectors, x, V, u, m, senders, modulated_params,
           W1, A1, B1, M1, W2, A2, B2, M2, W3, A3, B3, M3, W4, A4, B4, M4,
           Wlin):
    f32 = jnp.float32
    m1r = m[None, :]
    u1r = u[None, :]
    vec_t = vectors.T
    mod_t = modulated_params.T
    exp32 = jnp.asarray(_EXP32)

    c1 = 1.0 / math.sqrt(D_X)
    w1f = jnp.concatenate([W1[:, :MUL] * c1, (W1[:, MUL:] @ exp32) * c1],
                          axis=1)
    b1f = jnp.concatenate([B1[:, :MUL], B1[:, MUL:] @ exp32],
                          axis=1) * (c1 * ALPHA / RANK)
    kp_np = np.zeros((7 + MOD_DIM, 102), np.float32)
    kp_np[0, 96] = 1.0
    kp_np[1:4, 97] = 1.0
    kp_np[4:7, 0:96] = _TILE3
    kp = jnp.asarray(kp_np).at[7:, 98:102].set(M1)
    c2 = 1.0 / math.sqrt(D_X + 2 * MUL)
    w2 = W2 * c2
    b2 = B2 * (c2 * ALPHA / RANK)
    c3 = 1.0 / math.sqrt(HID)
    w3 = W3 * c3
    b3 = B3 * (c3 * ALPHA / RANK)
    w4 = W4 * c3
    b4 = B4 * (c3 * ALPHA / RANK)
    redp_np = np.zeros((C, 2 * MUL), np.float32)
    redp_np[:MUL, :MUL] = np.eye(MUL, dtype=np.float32)
    redp_np[MUL:, MUL:] = _RED3 * (1.0 / _SQ3)
    redp = jnp.asarray(redp_np)
    p1_np = np.zeros((C, 3 * MUL), np.float32)
    p1_np[:MUL, :] = _EXP32
    p1 = jnp.asarray(p1_np)
    eye3c = jnp.asarray(_EYE3)
    ce = 1.0 / math.sqrt(2 * MUL)
    wtop = jnp.kron(Wlin[:MUL, :], eye3c) * ce
    wbot = jnp.kron(Wlin[MUL:, :], eye3c) * ce
    ks_np = np.zeros((3 + MOD_DIM, 15), np.float32)
    ks_np[0, 0] = 1.0
    ks_np[1, 1] = 1.0
    ks_np[2, 2] = 1.0
    ks = jnp.asarray(ks_np)
    ks = ks.at[3:, 3:7].set(M2).at[3:, 7:11].set(M3).at[3:, 11:15].set(M4)

    vals = pl.pallas_call(
        _pre_kernel,
        grid=(GRID,),
        in_specs=[
            _edge_spec(D_X), _thin_spec(1), _thin_spec(3), _thin_spec(MOD_DIM),
            _full_spec(w1f.shape), _full_spec(A1.shape),
            _full_spec(b1f.shape), _full_spec(kp.shape),
        ],
        out_specs=_edge_spec(C),
        out_shape=jax.ShapeDtypeStruct((E, C), f32),
    )(x, m1r, vec_t, mod_t, w1f, A1, b1f, kp)

    snd = senders.astype(jnp.int32).reshape(NCHK, B)
    zb2 = jnp.zeros((B, C), f32)
    zb1 = jnp.zeros((B,), f32)
    summed, nnb = _seg_call()(vals, m, snd, zb2, zb1)
    nnb1r = nnb[None, :]

    x_out, v_out = pl.pallas_call(
        _post_kernel,
        grid=(GRID,),
        in_specs=[
            _edge_spec(D_X), _thin_spec(1), _thin_spec(1), _thin_spec(1),
            _edge_spec(C), _edge_spec(4 * MUL), _thin_spec(MOD_DIM),
            _full_spec((D_X, HID)), _full_spec((2 * MUL, HID)),
            _full_spec((D_X, RANK)), _full_spec((2 * MUL, RANK)),
            _full_spec(b2.shape),
            _full_spec(w3.shape), _full_spec(A3.shape), _full_spec(b3.shape),
            _full_spec(w4.shape), _full_spec(A4.shape), _full_spec(b4.shape),
            _full_spec(p1.shape), _full_spec(redp.shape),
            _full_spec(wtop.shape), _full_spec(wbot.shape),
            _full_spec(ks.shape),
        ],
        out_specs=[_edge_spec(D_X), _edge_spec(3 * MUL)],
        out_shape=[
            jax.ShapeDtypeStruct((E, D_X), f32),
            jax.ShapeDtypeStruct((E, 3 * MUL), f32),
        ],
    )(x, m1r, u1r, nnb1r, summed, V, mod_t,
      w2[:D_X], w2[D_X:], A2[:D_X], A2[D_X:], b2,
      w3, A3, b3, w4, A4, b4,
      p1, redp, wtop, wbot, ks)
    return x_out, v_out


# speedup vs baseline: 1.1969x; 1.1969x over previous
import functools
import math

import jax
import jax.numpy as jnp
import numpy as np
from jax import lax
from jax.experimental import pallas as pl
from jax.experimental.pallas import tpu as pltpu
from jax.experimental.pallas import tpu_sc as plsc

N_NODES = 10000
E = 160000
D_X = 128
MUL = 32
MOD_DIM = 16
RANK = 4
ALPHA = 8.0
HID = 128

C = 128
BE = 6400
GRID = E // BE

NC = 2
NT = 16
B = 128
NCHK = E // B
NPAIR = NCHK // 2
PE = 2 * B
ACC_N = 10240

_SQ3 = math.sqrt(3.0)
_D0 = (((0,), (0,)), ((), ()))

_EXP32 = np.zeros((MUL, 3 * MUL), np.float32)
_TILE3 = np.zeros((3, 3 * MUL), np.float32)
_RED3 = np.zeros((3 * MUL, MUL), np.float32)
for _c in range(MUL):
    for _i in range(3):
        _EXP32[_c, 3 * _c + _i] = 1.0
        _TILE3[_i, 3 * _c + _i] = 1.0
        _RED3[3 * _c + _i, _c] = 1.0
_K42 = np.array([[1, 0], [0, 1], [0, 1], [0, 1]], np.float32)
_EYE3 = np.eye(3, dtype=np.float32)


def _pre_kernel(x_ref, m_ref, vt_ref, md_ref, w1f_ref, a1_ref,
                b1f_ref, kp_ref, vals_ref):
    vt = vt_ref[...]
    stack = jnp.concatenate([m_ref[...], vt * vt, vt, md_ref[...]],
                            axis=0)
    cols = lax.dot_general(stack, kp_ref[...], _D0,
                           preferred_element_type=jnp.float32)
    vtil = cols[:, 0:96]
    m = cols[:, 96:97]
    scal = (_SQ3 * m) / (jnp.sqrt(cols[:, 97:98]) + 1e-12)
    s1 = cols[:, 98:102]
    xm = x_ref[...] * m
    t = jnp.dot(xm, a1_ref[...], preferred_element_type=jnp.float32) * s1
    w = (jnp.dot(xm, w1f_ref[...], preferred_element_type=jnp.float32)
         + jnp.dot(t, b1f_ref[...], preferred_element_type=jnp.float32))
    vals_ref[:, 0:MUL] = w[:, 0:MUL] * m
    vals_ref[:, MUL:C] = w[:, MUL:C] * vtil * scal


def _post_kernel(x_ref, m_ref, u_ref, nnb_ref, sm_ref, v_ref, md_ref,
                 w2a_ref, w2b_ref, a2a_ref, a2b_ref,
                 b2_ref,
                 w3_ref, a3_ref, b3_ref,
                 w4_ref, a4_ref, b4_ref,
                 p1_ref, redp_ref, wtop_ref, wbot_ref, ks_ref,
                 xout_ref, vout_ref):
    stack = jnp.concatenate([m_ref[...], u_ref[...], nnb_ref[...],
                             md_ref[...]], axis=0)
    cols = lax.dot_general(stack, ks_ref[...], _D0,
                           preferred_element_type=jnp.float32)
    m = cols[:, 0:1]
    u = cols[:, 1:2]
    inv = 1.0 / (cols[:, 2:3] + 1e-5)
    s2 = cols[:, 3:7]
    s3 = cols[:, 7:11]
    s4 = cols[:, 11:15]
    swvw = sm_ref[...] * inv
    v = v_ref[...]
    v_v = v[:, MUL:]
    pv = swvw * v
    out0ab = jnp.dot(pv, redp_ref[...], preferred_element_type=jnp.float32)
    sw_e = jnp.dot(swvw, p1_ref[...], preferred_element_type=jnp.float32)
    vs_e = jnp.dot(v, p1_ref[...], preferred_element_type=jnp.float32)
    vout = (jnp.dot(sw_e * v_v, wtop_ref[...], preferred_element_type=jnp.float32)
            + jnp.dot(swvw[:, MUL:] * vs_e, wbot_ref[...],
                      preferred_element_type=jnp.float32))
    xm = x_ref[...] * m

    t2 = (jnp.dot(xm, a2a_ref[...], preferred_element_type=jnp.float32)
          + jnp.dot(out0ab, a2b_ref[...], preferred_element_type=jnp.float32)
          ) * s2
    h = (jnp.dot(xm, w2a_ref[...], preferred_element_type=jnp.float32)
         + jnp.dot(out0ab, w2b_ref[...], preferred_element_type=jnp.float32)
         + jnp.dot(t2, b2_ref[...], preferred_element_type=jnp.float32))
    h = h * jax.nn.sigmoid(h)

    def lora(hh, w_r, a_r, b_r, s):
        t = jnp.dot(hh, a_r[...], preferred_element_type=jnp.float32) * s
        return (jnp.dot(hh, w_r[...], preferred_element_type=jnp.float32)
                + jnp.dot(t, b_r[...], preferred_element_type=jnp.float32))

    h = lora(h, w3_ref, a3_ref, b3_ref, s3)
    h = h * jax.nn.sigmoid(h)
    h = lora(h, w4_ref, a4_ref, b4_ref, s4)
    xout_ref[...] = u * h
    vout_ref[...] = vout


def _seg_body(vals_hbm, m_hbm, snd_hbm, zb2_hbm, zb1_hbm,
              out_hbm, outm_hbm,
              idx2, buf2, mbuf2, acc, accm,
              semi, semv, semm, semg, semw, semwm):
    tile = lax.axis_index("s")
    core = lax.axis_index("c")
    rpt = ACC_N // NT

    for k in range(rpt // B):
        pltpu.sync_copy(zb2_hbm, acc.at[pl.ds(tile * rpt + k * B, B)])
        pltpu.sync_copy(zb1_hbm, accm.at[pl.ds(tile * rpt + k * B, B)])
    plsc.subcore_barrier()

    base_p = NCHK // NT
    remp = NCHK - base_p * NT
    p0 = tile * base_p + jnp.minimum(tile, remp)
    nch = base_p + jnp.where(tile < remp, 1, 0)

    def start_loads(p, b):
        pltpu.async_copy(snd_hbm.at[p], idx2.at[b], semi.at[b])
        pltpu.async_copy(vals_hbm.at[pl.ds(p * B, B)], buf2.at[b], semv.at[b])
        pltpu.async_copy(m_hbm.at[pl.ds(p * B, B)], mbuf2.at[b], semm.at[b])

    start_loads(p0, 0)

    def sbody(g, carry):
        b = g % 2
        pltpu.make_async_copy(snd_hbm.at[0], idx2.at[b], semi.at[b]).wait()
        pltpu.make_async_copy(vals_hbm.at[pl.ds(0, B)], buf2.at[b],
                              semv.at[b]).wait()
        pltpu.make_async_copy(m_hbm.at[pl.ds(0, B)], mbuf2.at[b],
                              semm.at[b]).wait()
        pl.when(g + 1 < nch)(lambda: start_loads(p0 + g + 1, (g + 1) % 2))
        pltpu.sync_copy(buf2.at[b], acc.at[idx2.at[b]], add=True)
        pltpu.sync_copy(mbuf2.at[b], accm.at[idx2.at[b]], add=True)
        return carry

    lax.fori_loop(0, nch, sbody, 0)
    plsc.subcore_barrier()

    w = core * NT + tile
    nw = NC * NT
    base_q = NCHK // nw
    rem = NCHK - base_q * nw
    q0 = w * base_q + jnp.minimum(w, rem)
    nq = base_q + jnp.where(w < rem, 1, 0)

    pltpu.async_copy(snd_hbm.at[q0], idx2.at[0], semi.at[0])

    def gbody(g, carry):
        b = g % 2
        p = q0 + g
        pltpu.make_async_copy(snd_hbm.at[0], idx2.at[b], semi.at[b]).wait()

        def fetch_next():
            pltpu.async_copy(snd_hbm.at[p + 1], idx2.at[(g + 1) % 2],
                             semi.at[(g + 1) % 2])

        pl.when(g + 1 < nq)(fetch_next)

        def wait_writes():
            pltpu.make_async_copy(buf2.at[b], out_hbm.at[pl.ds(0, B)],
                                  semw.at[b]).wait()
            pltpu.make_async_copy(mbuf2.at[b], outm_hbm.at[pl.ds(0, B)],
                                  semwm.at[b]).wait()

        pl.when(g >= 2)(wait_writes)
        pltpu.async_copy(acc.at[idx2.at[b]], buf2.at[b], semg).wait()
        pltpu.async_copy(accm.at[idx2.at[b]], mbuf2.at[b], semg).wait()
        pltpu.async_copy(buf2.at[b], out_hbm.at[pl.ds(p * B, B)], semw.at[b])
        pltpu.async_copy(mbuf2.at[b], outm_hbm.at[pl.ds(p * B, B)],
                         semwm.at[b])
        return carry

    lax.fori_loop(0, nq, gbody, 0)
    for b in range(2):
        pltpu.make_async_copy(buf2.at[b], out_hbm.at[pl.ds(0, B)],
                              semw.at[b]).wait()
        pltpu.make_async_copy(mbuf2.at[b], outm_hbm.at[pl.ds(0, B)],
                              semwm.at[b]).wait()


@functools.cache
def _seg_call():
    return functools.partial(
        pl.kernel,
        out_type=[
            jax.ShapeDtypeStruct((E, C), jnp.float32),
            jax.ShapeDtypeStruct((E,), jnp.float32),
        ],
        mesh=plsc.VectorSubcoreMesh(core_axis_name="c", subcore_axis_name="s",
                                    num_cores=NC, num_subcores=NT),
        scratch_types=[
            pltpu.VMEM((2, B), jnp.int32),
            pltpu.VMEM((2, B, C), jnp.float32),
            pltpu.VMEM((2, B), jnp.float32),
            pltpu.VMEM_SHARED((ACC_N, C), jnp.float32),
            pltpu.VMEM_SHARED((ACC_N,), jnp.float32),
            pltpu.SemaphoreType.DMA((2,)),
            pltpu.SemaphoreType.DMA((2,)),
            pltpu.SemaphoreType.DMA((2,)),
            pltpu.SemaphoreType.DMA,
            pltpu.SemaphoreType.DMA((2,)),
            pltpu.SemaphoreType.DMA((2,)),
        ],
    )(_seg_body)


def _edge_spec(d):
    return pl.BlockSpec((BE, d), lambda i: (i, 0))


def _thin_spec(k):
    return pl.BlockSpec((k, BE), lambda i: (0, i))


def _full_spec(shape):
    nd = len(shape)
    return pl.BlockSpec(shape, lambda i: (0,) * nd)


def kernel(vectors, x, V, u, m, senders, modulated_params,
           W1, A1, B1, M1, W2, A2, B2, M2, W3, A3, B3, M3, W4, A4, B4, M4,
           Wlin):
    f32 = jnp.float32
    m1r = m[None, :]
    u1r = u[None, :]
    vec_t = vectors.T
    mod_t = modulated_params.T
    exp32 = jnp.asarray(_EXP32)

    c1 = 1.0 / math.sqrt(D_X)
    w1f = jnp.concatenate([W1[:, :MUL] * c1, (W1[:, MUL:] @ exp32) * c1],
                          axis=1)
    b1f = jnp.concatenate([B1[:, :MUL], B1[:, MUL:] @ exp32],
                          axis=1) * (c1 * ALPHA / RANK)
    kp_np = np.zeros((7 + MOD_DIM, 102), np.float32)
    kp_np[0, 96] = 1.0
    kp_np[1:4, 97] = 1.0
    kp_np[4:7, 0:96] = _TILE3
    kp = jnp.asarray(kp_np).at[7:, 98:102].set(M1)
    c2 = 1.0 / math.sqrt(D_X + 2 * MUL)
    w2 = W2 * c2
    b2 = B2 * (c2 * ALPHA / RANK)
    c3 = 1.0 / math.sqrt(HID)
    w3 = W3 * c3
    b3 = B3 * (c3 * ALPHA / RANK)
    w4 = W4 * c3
    b4 = B4 * (c3 * ALPHA / RANK)
    redp_np = np.zeros((C, 2 * MUL), np.float32)
    redp_np[:MUL, :MUL] = np.eye(MUL, dtype=np.float32)
    redp_np[MUL:, MUL:] = _RED3 * (1.0 / _SQ3)
    redp = jnp.asarray(redp_np)
    p1_np = np.zeros((C, 3 * MUL), np.float32)
    p1_np[:MUL, :] = _EXP32
    p1 = jnp.asarray(p1_np)
    eye3c = jnp.asarray(_EYE3)
    ce = 1.0 / math.sqrt(2 * MUL)
    wtop = jnp.kron(Wlin[:MUL, :], eye3c) * ce
    wbot = jnp.kron(Wlin[MUL:, :], eye3c) * ce
    ks_np = np.zeros((3 + MOD_DIM, 15), np.float32)
    ks_np[0, 0] = 1.0
    ks_np[1, 1] = 1.0
    ks_np[2, 2] = 1.0
    ks = jnp.asarray(ks_np)
    ks = ks.at[3:, 3:7].set(M2).at[3:, 7:11].set(M3).at[3:, 11:15].set(M4)

    vals = pl.pallas_call(
        _pre_kernel,
        grid=(GRID,),
        in_specs=[
            _edge_spec(D_X), _thin_spec(1), _thin_spec(3), _thin_spec(MOD_DIM),
            _full_spec(w1f.shape), _full_spec(A1.shape),
            _full_spec(b1f.shape), _full_spec(kp.shape),
        ],
        out_specs=_edge_spec(C),
        out_shape=jax.ShapeDtypeStruct((E, C), f32),
    )(x, m1r, vec_t, mod_t, w1f, A1, b1f, kp)

    snd = senders.astype(jnp.int32).reshape(NCHK, B)
    zb2 = jnp.zeros((B, C), f32)
    zb1 = jnp.zeros((B,), f32)
    summed, nnb = _seg_call()(vals, m, snd, zb2, zb1)
    nnb1r = nnb[None, :]

    x_out, v_out = pl.pallas_call(
        _post_kernel,
        grid=(GRID,),
        in_specs=[
            _edge_spec(D_X), _thin_spec(1), _thin_spec(1), _thin_spec(1),
            _edge_spec(C), _edge_spec(4 * MUL), _thin_spec(MOD_DIM),
            _full_spec((D_X, HID)), _full_spec((2 * MUL, HID)),
            _full_spec((D_X, RANK)), _full_spec((2 * MUL, RANK)),
            _full_spec(b2.shape),
            _full_spec(w3.shape), _full_spec(A3.shape), _full_spec(b3.shape),
            _full_spec(w4.shape), _full_spec(A4.shape), _full_spec(b4.shape),
            _full_spec(p1.shape), _full_spec(redp.shape),
            _full_spec(wtop.shape), _full_spec(wbot.shape),
            _full_spec(ks.shape),
        ],
        out_specs=[_edge_spec(D_X), _edge_spec(3 * MUL)],
        out_shape=[
            jax.ShapeDtypeStruct((E, D_X), f32),
            jax.ShapeDtypeStruct((E, 3 * MUL), f32),
        ],
    )(x, m1r, u1r, nnb1r, summed, V, mod_t,
      w2[:D_X], w2[D_X:], A2[:D_X], A2[D_X:], b2,
      w3, A3, b3, w4, A4, b4,
      p1, redp, wtop, wbot, ks)
    return x_out, v_out
